# baseline (speedup 1.0000x reference)
import functools

import jax
import jax.numpy as jnp
from jax import lax
from jax.experimental import pallas as pl
from jax.experimental.pallas import tpu as pltpu
from jax.experimental.pallas import tpu_sc as plsc

N = 10000
NG = 64
F1 = 128
H = 256
HH = 128
E_RAW = 320000
E_TOT = E_RAW + N
TILES = 16
B64 = 64
GRP = 8
NGRP = 42
NPAIR = NGRP // 2
NB64 = GRP * NGRP
CHUNK = NB64 * B64
E_PAD = TILES * CHUNK
DEN_PAD = 10240
NEG = -1e30

_f32 = jnp.float32
_i32 = jnp.int32


def _dot(a, b):
  return jnp.dot(a, b, precision=lax.Precision.HIGHEST,
                 preferred_element_type=_f32)


RB = 1000
NRB = N // RB


def _tc_layer1_body(x_ref, w_ref, as_ref, ad_ref,
                    h0_ref, h1_ref, asv_ref, adv_ref):
  h = _dot(x_ref[...], w_ref[...])
  h0_ref[...] = h[:, :HH]
  h1_ref[...] = h[:, HH:]
  asv_ref[...] = _dot(h, as_ref[...].reshape(H, 1)).reshape(1, 1, RB)
  adv_ref[...] = _dot(h, ad_ref[...].reshape(H, 1)).reshape(1, 1, RB)


def _tc_layer_body(o0_ref, o1_ref, b_ref, w_ref, as_ref, ad_ref,
                   h0_ref, h1_ref, asv_ref, adv_ref):
  x0 = jax.nn.relu(o0_ref[...] + b_ref[...][:, :HH])
  x1 = jax.nn.relu(o1_ref[...] + b_ref[...][:, HH:])
  h = _dot(x0, w_ref[...][:HH, :]) + _dot(x1, w_ref[...][HH:, :])
  h0_ref[...] = h[:, :HH]
  h1_ref[...] = h[:, HH:]
  asv_ref[...] = _dot(h, as_ref[...].reshape(H, 1)).reshape(1, 1, RB)
  adv_ref[...] = _dot(h, ad_ref[...].reshape(H, 1)).reshape(1, 1, RB)


def _tc_readout_body(o0_ref, o1_ref, b_ref, batch_ref,
                     fw1_ref, fb1_ref, fw2_ref, fb2_ref, out_ref):
  x0 = jax.nn.relu(o0_ref[...] + b_ref[...][:, :HH])
  x1 = jax.nn.relu(o1_ref[...] + b_ref[...][:, HH:])
  gids = lax.broadcasted_iota(_i32, (NG, N), 0)
  onehot = (gids == batch_ref[...]).astype(_f32)
  s0 = _dot(onehot, x0)
  s1 = _dot(onehot, x1)
  cnt = jnp.sum(onehot, axis=1, keepdims=True)
  inv = 1.0 / jnp.maximum(cnt, 1.0)
  fw1 = fw1_ref[...]
  z = jax.nn.relu(_dot(s0 * inv, fw1[:HH, :]) + _dot(s1 * inv, fw1[HH:, :])
                  + fb1_ref[...])
  out_ref[...] = _dot(z, fw2_ref[...]) + fb2_ref[...]


_row_block = pl.BlockSpec((RB, HH), lambda i: (i, 0))
_vec_block = pl.BlockSpec((1, 1, RB), lambda i: (i, 0, 0))
_layer_out_specs = [_row_block, _row_block, _vec_block, _vec_block]
_layer_out_shape = [
    jax.ShapeDtypeStruct((N, HH), _f32),
    jax.ShapeDtypeStruct((N, HH), _f32),
    jax.ShapeDtypeStruct((NRB, 1, RB), _f32),
    jax.ShapeDtypeStruct((NRB, 1, RB), _f32),
]

_layer1_call = pl.pallas_call(
    _tc_layer1_body,
    grid=(NRB,),
    in_specs=[
        pl.BlockSpec((RB, F1), lambda i: (i, 0)),
        pl.BlockSpec((F1, H), lambda i: (0, 0)),
        pl.BlockSpec((1, H), lambda i: (0, 0)),
        pl.BlockSpec((1, H), lambda i: (0, 0)),
    ],
    out_specs=_layer_out_specs,
    out_shape=_layer_out_shape,
)

_layer_call = pl.pallas_call(
    _tc_layer_body,
    grid=(NRB,),
    in_specs=[
        _row_block,
        _row_block,
        pl.BlockSpec((1, H), lambda i: (0, 0)),
        pl.BlockSpec((H, H), lambda i: (0, 0)),
        pl.BlockSpec((1, H), lambda i: (0, 0)),
        pl.BlockSpec((1, H), lambda i: (0, 0)),
    ],
    out_specs=_layer_out_specs,
    out_shape=_layer_out_shape,
)

_readout_call = pl.pallas_call(
    _tc_readout_body,
    out_shape=jax.ShapeDtypeStruct((NG, 10), _f32),
)


def _sc_body(asv_hbm, adv_hbm, src_hbm, dst_hbm,
             h0_hbm, h1_hbm, o0_hbm, o1_hbm,
             as_v, ad_v, rows0, rows1,
             src64a, src64b, dst64a, dst64b, attb, denb,
             zden, mscr, maxv,
             st0, st1, sg0, sg1, sd0, sd1, ss0, ss1, sp,
             out_sh, den_sh, max_sh):
  c = lax.axis_index("c")
  s = lax.axis_index("s")
  lanes = lax.iota(_i32, 16)
  base = s * CHUNK
  srcb = (src64a, src64b)
  dstb = (dst64a, dst64b)
  stsem = (st0, st1)

  pltpu.sync_copy(asv_hbm, as_v)
  pltpu.sync_copy(adv_hbm, ad_v)

  def _zrow(j, carry):
    for f in range(8):
      rows0[j, pl.ds(16 * f, 16)] = jnp.zeros((16,), _f32)
    return carry
  lax.fori_loop(0, B64, _zrow, 0)

  def _zden(i, carry):
    zden[pl.ds(16 * i, 16)] = jnp.zeros((16,), _f32)
    return carry
  lax.fori_loop(0, 40, _zden, 0)

  pltpu.sync_copy(zden, den_sh.at[pl.ds(s * 640, 640)])

  def _row_chunks(fn):
    @pl.when(s < 15)
    def _():
      for off, sz in [(64 * k, 64) for k in range(9)] + [(576, 56)]:
        fn(s * 632 + off, sz)

    @pl.when(s == 15)
    def _():
      for off, sz in [(64 * k, 64) for k in range(8)] + [(512, 8)]:
        fn(9480 + off, sz)

  def _zero_out(r0, sz):
    pltpu.sync_copy(rows0.at[pl.ds(0, sz)], out_sh.at[pl.ds(r0, sz)])
  _row_chunks(_zero_out)

  def _off(g2, p):
    return pl.multiple_of((2 * g2 + p) * GRP, GRP)

  def _issue_stage(g2, p):
    off = _off(g2, p)
    pltpu.async_copy(src_hbm.at[s, pl.ds(off, GRP)], srcb[p], stsem[p])
    pltpu.async_copy(dst_hbm.at[s, pl.ds(off, GRP)], dstb[p], stsem[p])

  def _wait_stage(g2, p):
    off = _off(g2, p)
    pltpu.make_async_copy(
        src_hbm.at[s, pl.ds(off, GRP)], srcb[p], stsem[p]).wait()
    pltpu.make_async_copy(
        dst_hbm.at[s, pl.ds(off, GRP)], dstb[p], stsem[p]).wait()

  def _stage_next(g2, p):
    if p == 0:
      _issue_stage(g2, 1)
    else:
      @pl.when(g2 < NPAIR - 1)
      def _():
        _issue_stage(g2 + 1, 0)

  def _alpha64(g, p, r, i):
    sl = pl.ds(16 * i, 16)
    a = (plsc.load_gather(as_v, [srcb[p][r, sl]])
         + plsc.load_gather(ad_v, [dstb[p][r, sl]]))
    a = jnp.maximum(a, 0.2 * a)
    flat = base + (g * GRP + r) * B64 + i * 16 + lanes
    return jnp.where(flat < E_TOT, a, NEG)

  _issue_stage(0, 0)

  def _max_pair(g2, mvec):
    for p in range(2):
      _wait_stage(g2, p)
      _stage_next(g2, p)
      g = 2 * g2 + p

      def _max_r(r, mv, _p=p, _g=g):
        def _max_i(i, m2):
          return jnp.maximum(m2, _alpha64(_g, _p, r, i))
        return lax.fori_loop(0, 4, _max_i, mv)
      mvec = lax.fori_loop(0, GRP, _max_r, mvec)
    return mvec

  mvec = lax.fori_loop(0, NPAIR, _max_pair, jnp.full((16,), NEG, _f32))
  mscr[...] = mvec
  pltpu.sync_copy(mscr, max_sh.at[s])
  plsc.subcore_barrier()

  pltpu.sync_copy(max_sh, maxv)

  def _mred(i, mv):
    return jnp.maximum(mv, maxv[i, :])
  gmax = jnp.max(lax.fori_loop(0, TILES, _mred, jnp.full((16,), NEG, _f32)))

  _issue_stage(0, 0)

  def _den_pair(g2, carry):
    for p in range(2):
      _wait_stage(g2, p)
      _stage_next(g2, p)
      g = 2 * g2 + p
      descs = []
      for r in range(GRP):
        def _den_i(i, c3, _r=r, _p=p, _g=g):
          attb[_r, pl.ds(16 * i, 16)] = jnp.exp(_alpha64(_g, _p, _r, i)
                                                - gmax)
          return c3
        lax.fori_loop(0, 4, _den_i, 0)
        descs.append(
            pltpu.async_copy(attb.at[r], den_sh.at[dstb[p].at[r]], sp,
                             add=True))
      for d in descs:
        d.wait()
    return carry
  lax.fori_loop(0, NPAIR, _den_pair, 0)
  plsc.subcore_barrier()

  def _aggregate(h_hbm, out_hbm):
    bufs = (rows0, rows1)
    gsem = (sg0, sg1)
    dsem = (sd0, sd1)
    ssem = (ss0, ss1)

    def _drain_sca(pp):
      pltpu.make_async_copy(
          bufs[pp], out_sh.at[dstb[0].at[0]], ssem[pp]).wait()

    def _drain_tail():
      _drain_sca(0)
      _drain_sca(1)

    _issue_stage(0, 0)

    def _agg_pair(g2, carry):
      for p in range(2):
        g = 2 * g2 + p
        if p == 1:
          _drain_tail()
        else:
          @pl.when(g2 > 0)
          def _():
            _drain_tail()
        _wait_stage(g2, p)
        _stage_next(g2, p)

        gat = {}
        den = {}
        sca = {}
        gat[0] = pltpu.async_copy(h_hbm.at[srcb[p].at[0]], bufs[0], gsem[0])
        den[0] = pltpu.async_copy(den_sh.at[dstb[p].at[0]], denb.at[0],
                                  dsem[0])
        for r in range(GRP):
          pp = r % 2
          if r + 1 < GRP:
            q = (r + 1) % 2
            if r >= 1:
              sca[r - 1].wait()
            gat[r + 1] = pltpu.async_copy(
                h_hbm.at[srcb[p].at[r + 1]], bufs[q], gsem[q])
            den[r + 1] = pltpu.async_copy(
                den_sh.at[dstb[p].at[r + 1]], denb.at[r + 1], dsem[q])
          gat[r].wait()
          den[r].wait()

          def _att_i(i, c3, _r=r, _p=p, _g=g):
            sl = pl.ds(16 * i, 16)
            ex = jnp.exp(_alpha64(_g, _p, _r, i) - gmax)
            attb[_r, sl] = ex / (denb[_r, sl] + 1e-16)
            return c3
          lax.fori_loop(0, 4, _att_i, 0)

          def _scale_e(e, c3, _pp=pp, _r=r):
            w = plsc.load_gather(
                attb, [jnp.full((16,), _r, _i32), jnp.full((16,), e, _i32)])
            for f in range(8):
              slf = pl.ds(16 * f, 16)
              bufs[_pp][e, slf] = bufs[_pp][e, slf] * w
            return c3
          lax.fori_loop(0, B64, _scale_e, 0)

          sca[r] = pltpu.async_copy(
              bufs[pp], out_sh.at[dstb[p].at[r]], ssem[pp], add=True)
      return carry
    lax.fori_loop(0, NPAIR, _agg_pair, 0)
    _drain_tail()
    plsc.subcore_barrier()

    def _write_out(r0, sz):
      pltpu.sync_copy(out_sh.at[pl.ds(r0, sz)], rows0.at[pl.ds(0, sz)])
      pltpu.sync_copy(rows0.at[pl.ds(0, sz)], out_hbm.at[pl.ds(r0, sz)])
    _row_chunks(_write_out)

  @pl.when(c == 0)
  def _():
    _aggregate(h0_hbm, o0_hbm)

  @pl.when(c == 1)
  def _():
    _aggregate(h1_hbm, o1_hbm)


_sc_call = functools.partial(
    pl.kernel,
    mesh=plsc.VectorSubcoreMesh(core_axis_name="c", subcore_axis_name="s"),
    compiler_params=pltpu.CompilerParams(needs_layout_passes=False),
    out_type=[
        jax.ShapeDtypeStruct((N, HH), _f32),
        jax.ShapeDtypeStruct((N, HH), _f32),
    ],
    scratch_types=[
        pltpu.VMEM((N,), _f32),
        pltpu.VMEM((N,), _f32),
        pltpu.VMEM((B64, HH), _f32),
        pltpu.VMEM((B64, HH), _f32),
        pltpu.VMEM((GRP, B64), _i32),
        pltpu.VMEM((GRP, B64), _i32),
        pltpu.VMEM((GRP, B64), _i32),
        pltpu.VMEM((GRP, B64), _i32),
        pltpu.VMEM((GRP, B64), _f32),
        pltpu.VMEM((GRP, B64), _f32),
        pltpu.VMEM((640,), _f32),
        pltpu.VMEM((16,), _f32),
        pltpu.VMEM((16, 16), _f32),
        pltpu.SemaphoreType.DMA,
        pltpu.SemaphoreType.DMA,
        pltpu.SemaphoreType.DMA,
        pltpu.SemaphoreType.DMA,
        pltpu.SemaphoreType.DMA,
        pltpu.SemaphoreType.DMA,
        pltpu.SemaphoreType.DMA,
        pltpu.SemaphoreType.DMA,
        pltpu.SemaphoreType.DMA,
        pltpu.VMEM_SHARED((N, HH), _f32),
        pltpu.VMEM_SHARED((DEN_PAD,), _f32),
        pltpu.VMEM_SHARED((16, 16), _f32),
    ],
)(_sc_body)


def kernel(x, edge_index, batch, W1, a_s1, a_d1, b1, W2, a_s2, a_d2, b2,
           W3, a_s3, a_d3, b3, fcW1, fcb1, fcW2, fcb2):
  loop = jnp.arange(N, dtype=_i32)
  pad = jnp.zeros((E_PAD - E_TOT,), _i32)
  srcf = jnp.concatenate([edge_index[0].astype(_i32), loop, pad])
  dstf = jnp.concatenate([edge_index[1].astype(_i32), loop, pad])
  src = srcf.reshape(TILES, NB64, B64)
  dst = dstf.reshape(TILES, NB64, B64)

  h0, h1, asv, adv = _layer1_call(x, W1, a_s1.reshape(1, H),
                                  a_d1.reshape(1, H))
  o0, o1 = _sc_call(asv.reshape(N), adv.reshape(N), src, dst, h0, h1)

  h0, h1, asv, adv = _layer_call(o0, o1, b1.reshape(1, H), W2,
                                 a_s2.reshape(1, H), a_d2.reshape(1, H))
  o0, o1 = _sc_call(asv.reshape(N), adv.reshape(N), src, dst, h0, h1)

  h0, h1, asv, adv = _layer_call(o0, o1, b2.reshape(1, H), W3,
                                 a_s3.reshape(1, H), a_d3.reshape(1, H))
  o0, o1 = _sc_call(asv.reshape(N), adv.reshape(N), src, dst, h0, h1)

  out = _readout_call(o0, o1, b3.reshape(1, H), batch.reshape(1, N),
                      fcW1, fcb1.reshape(1, 196), fcW2, fcb2.reshape(1, 10))
  return out


# speedup vs baseline: 1.0043x; 1.0043x over previous
import functools

import jax
import jax.numpy as jnp
from jax import lax
from jax.experimental import pallas as pl
from jax.experimental.pallas import tpu as pltpu
from jax.experimental.pallas import tpu_sc as plsc

N = 10000
NG = 64
F1 = 128
H = 256
HH = 128
E_RAW = 320000
E_TOT = E_RAW + N
TILES = 16
BLK = 128
GRP = 8
NGRP = 21
NBLK = GRP * NGRP
E_PAD = TILES * NBLK * BLK
CHUNK = NBLK * BLK
B64 = 64
NB64 = CHUNK // B64
NGRP64 = NB64 // GRP
DEN_PAD = 10240
NEG = -1e30

_f32 = jnp.float32
_i32 = jnp.int32


def _dot(a, b):
  return jnp.dot(a, b, precision=lax.Precision.HIGHEST,
                 preferred_element_type=_f32)


RB = 1000
NRB = N // RB


def _tc_layer1_body(x_ref, w_ref, as_ref, ad_ref,
                    h0_ref, h1_ref, asv_ref, adv_ref):
  h = _dot(x_ref[...], w_ref[...])
  h0_ref[...] = h[:, :HH]
  h1_ref[...] = h[:, HH:]
  asv_ref[...] = _dot(h, as_ref[...].reshape(H, 1)).reshape(1, 1, RB)
  adv_ref[...] = _dot(h, ad_ref[...].reshape(H, 1)).reshape(1, 1, RB)


def _tc_layer_body(o0_ref, o1_ref, b_ref, w_ref, as_ref, ad_ref,
                   h0_ref, h1_ref, asv_ref, adv_ref):
  x0 = jax.nn.relu(o0_ref[...] + b_ref[...][:, :HH])
  x1 = jax.nn.relu(o1_ref[...] + b_ref[...][:, HH:])
  h = _dot(x0, w_ref[...][:HH, :]) + _dot(x1, w_ref[...][HH:, :])
  h0_ref[...] = h[:, :HH]
  h1_ref[...] = h[:, HH:]
  asv_ref[...] = _dot(h, as_ref[...].reshape(H, 1)).reshape(1, 1, RB)
  adv_ref[...] = _dot(h, ad_ref[...].reshape(H, 1)).reshape(1, 1, RB)


def _tc_readout_body(o0_ref, o1_ref, b_ref, batch_ref,
                     fw1_ref, fb1_ref, fw2_ref, fb2_ref, out_ref):
  x0 = jax.nn.relu(o0_ref[...] + b_ref[...][:, :HH])
  x1 = jax.nn.relu(o1_ref[...] + b_ref[...][:, HH:])
  gids = lax.broadcasted_iota(_i32, (NG, N), 0)
  onehot = (gids == batch_ref[...]).astype(_f32)
  s0 = _dot(onehot, x0)
  s1 = _dot(onehot, x1)
  cnt = jnp.sum(onehot, axis=1, keepdims=True)
  inv = 1.0 / jnp.maximum(cnt, 1.0)
  fw1 = fw1_ref[...]
  z = jax.nn.relu(_dot(s0 * inv, fw1[:HH, :]) + _dot(s1 * inv, fw1[HH:, :])
                  + fb1_ref[...])
  out_ref[...] = _dot(z, fw2_ref[...]) + fb2_ref[...]


_row_block = pl.BlockSpec((RB, HH), lambda i: (i, 0))
_vec_block = pl.BlockSpec((1, 1, RB), lambda i: (i, 0, 0))
_layer_out_specs = [_row_block, _row_block, _vec_block, _vec_block]
_layer_out_shape = [
    jax.ShapeDtypeStruct((N, HH), _f32),
    jax.ShapeDtypeStruct((N, HH), _f32),
    jax.ShapeDtypeStruct((NRB, 1, RB), _f32),
    jax.ShapeDtypeStruct((NRB, 1, RB), _f32),
]

_layer1_call = pl.pallas_call(
    _tc_layer1_body,
    grid=(NRB,),
    in_specs=[
        pl.BlockSpec((RB, F1), lambda i: (i, 0)),
        pl.BlockSpec((F1, H), lambda i: (0, 0)),
        pl.BlockSpec((1, H), lambda i: (0, 0)),
        pl.BlockSpec((1, H), lambda i: (0, 0)),
    ],
    out_specs=_layer_out_specs,
    out_shape=_layer_out_shape,
)

_layer_call = pl.pallas_call(
    _tc_layer_body,
    grid=(NRB,),
    in_specs=[
        _row_block,
        _row_block,
        pl.BlockSpec((1, H), lambda i: (0, 0)),
        pl.BlockSpec((H, H), lambda i: (0, 0)),
        pl.BlockSpec((1, H), lambda i: (0, 0)),
        pl.BlockSpec((1, H), lambda i: (0, 0)),
    ],
    out_specs=_layer_out_specs,
    out_shape=_layer_out_shape,
)

_readout_call = pl.pallas_call(
    _tc_readout_body,
    out_shape=jax.ShapeDtypeStruct((NG, 10), _f32),
)


def _sc_body(asv_hbm, adv_hbm, src_hbm, dst_hbm, src64_hbm, dst64_hbm,
             h0_hbm, h1_hbm, o0_hbm, o1_hbm,
             as_v, ad_v, rows0, rows1, src_st, dst_st, exb,
             src64, dst64, attb, denb, zden, mscr, maxv,
             sg0, sg1, sd0, sd1, ss0, ss1, sp,
             out_sh, den_sh, max_sh):
  c = lax.axis_index("c")
  s = lax.axis_index("s")
  lanes = lax.iota(_i32, 16)
  base = s * CHUNK

  pltpu.sync_copy(asv_hbm, as_v)
  pltpu.sync_copy(adv_hbm, ad_v)

  def _zrow(j, carry):
    for f in range(8):
      rows0[j, pl.ds(16 * f, 16)] = jnp.zeros((16,), _f32)
    return carry
  lax.fori_loop(0, B64, _zrow, 0)

  def _zden(i, carry):
    zden[pl.ds(16 * i, 16)] = jnp.zeros((16,), _f32)
    return carry
  lax.fori_loop(0, 40, _zden, 0)

  pltpu.sync_copy(zden, den_sh.at[pl.ds(s * 640, 640)])

  def _row_chunks(fn):
    @pl.when(s < 15)
    def _():
      for off, sz in [(64 * k, 64) for k in range(9)] + [(576, 56)]:
        fn(s * 632 + off, sz)

    @pl.when(s == 15)
    def _():
      for off, sz in [(64 * k, 64) for k in range(8)] + [(512, 8)]:
        fn(9480 + off, sz)

  def _zero_out(r0, sz):
    pltpu.sync_copy(rows0.at[pl.ds(0, sz)], out_sh.at[pl.ds(r0, sz)])
  _row_chunks(_zero_out)

  def _stage_idx(g):
    off = pl.multiple_of(g * GRP, GRP)
    pltpu.sync_copy(src_hbm.at[s, pl.ds(off, GRP)], src_st)
    pltpu.sync_copy(dst_hbm.at[s, pl.ds(off, GRP)], dst_st)

  def _alpha_vec(g, r, i):
    sl = pl.ds(16 * i, 16)
    sv = src_st[r, sl]
    dv = dst_st[r, sl]
    a = plsc.load_gather(as_v, [sv]) + plsc.load_gather(ad_v, [dv])
    a = jnp.maximum(a, 0.2 * a)
    flat = base + (g * GRP + r) * BLK + i * 16 + lanes
    return jnp.where(flat < E_TOT, a, NEG)

  def _max_g(g, mvec):
    _stage_idx(g)
    def _max_r(r, mv):
      def _max_i(i, m2):
        return jnp.maximum(m2, _alpha_vec(g, r, i))
      return lax.fori_loop(0, 8, _max_i, mv)
    return lax.fori_loop(0, GRP, _max_r, mvec)

  mvec = lax.fori_loop(0, NGRP, _max_g, jnp.full((16,), NEG, _f32))
  mscr[...] = mvec
  pltpu.sync_copy(mscr, max_sh.at[s])
  plsc.subcore_barrier()

  pltpu.sync_copy(max_sh, maxv)

  def _mred(i, mv):
    return jnp.maximum(mv, maxv[i, :])
  gmax = jnp.max(lax.fori_loop(0, TILES, _mred, jnp.full((16,), NEG, _f32)))

  def _den_g(g, carry):
    _stage_idx(g)
    descs = []
    for r in range(GRP):
      def _den_i(i, c3, _r=r):
        exb[_r, pl.ds(16 * i, 16)] = jnp.exp(_alpha_vec(g, _r, i) - gmax)
        return c3
      lax.fori_loop(0, 8, _den_i, 0)
      descs.append(
          pltpu.async_copy(exb.at[r], den_sh.at[dst_st.at[r]], sp, add=True))
    for d in descs:
      d.wait()
    return carry
  lax.fori_loop(0, NGRP, _den_g, 0)
  plsc.subcore_barrier()

  def _alpha64(g, r, i):
    sl = pl.ds(16 * i, 16)
    sv = src64[r, sl]
    dv = dst64[r, sl]
    a = plsc.load_gather(as_v, [sv]) + plsc.load_gather(ad_v, [dv])
    a = jnp.maximum(a, 0.2 * a)
    flat = base + (g * GRP + r) * B64 + i * 16 + lanes
    return jnp.where(flat < E_TOT, a, NEG)

  def _aggregate(h_hbm, out_hbm):
    bufs = (rows0, rows1)
    gsem = (sg0, sg1)
    dsem = (sd0, sd1)
    ssem = (ss0, ss1)

    def _agg_g(g, carry):
      off = pl.multiple_of(g * GRP, GRP)
      pltpu.sync_copy(src64_hbm.at[s, pl.ds(off, GRP)], src64)
      pltpu.sync_copy(dst64_hbm.at[s, pl.ds(off, GRP)], dst64)

      gat = {}
      den = {}
      sca = {}
      gat[0] = pltpu.async_copy(h_hbm.at[src64.at[0]], bufs[0], gsem[0])
      den[0] = pltpu.async_copy(den_sh.at[dst64.at[0]], denb.at[0], dsem[0])
      for r in range(GRP):
        p = r % 2
        if r + 1 < GRP:
          q = (r + 1) % 2
          if r >= 1:
            sca[r - 1].wait()
          gat[r + 1] = pltpu.async_copy(
              h_hbm.at[src64.at[r + 1]], bufs[q], gsem[q])
          den[r + 1] = pltpu.async_copy(
              den_sh.at[dst64.at[r + 1]], denb.at[r + 1], dsem[q])
        gat[r].wait()
        den[r].wait()

        def _att_i(i, c3, _r=r):
          sl = pl.ds(16 * i, 16)
          ex = jnp.exp(_alpha64(g, _r, i) - gmax)
          attb[_r, sl] = ex / (denb[_r, sl] + 1e-16)
          return c3
        lax.fori_loop(0, 4, _att_i, 0)

        def _scale_e(e, c3, _p=p, _r=r):
          w = plsc.load_gather(
              attb, [jnp.full((16,), _r, _i32), jnp.full((16,), e, _i32)])
          for f in range(8):
            slf = pl.ds(16 * f, 16)
            bufs[_p][e, slf] = bufs[_p][e, slf] * w
          return c3
        lax.fori_loop(0, B64, _scale_e, 0)

        sca[r] = pltpu.async_copy(
            bufs[p], out_sh.at[dst64.at[r]], ssem[p], add=True)
      sca[GRP - 2].wait()
      sca[GRP - 1].wait()
      return carry
    lax.fori_loop(0, NGRP64, _agg_g, 0)
    plsc.subcore_barrier()

    def _write_out(r0, sz):
      pltpu.sync_copy(out_sh.at[pl.ds(r0, sz)], rows0.at[pl.ds(0, sz)])
      pltpu.sync_copy(rows0.at[pl.ds(0, sz)], out_hbm.at[pl.ds(r0, sz)])
    _row_chunks(_write_out)

  @pl.when(c == 0)
  def _():
    _aggregate(h0_hbm, o0_hbm)

  @pl.when(c == 1)
  def _():
    _aggregate(h1_hbm, o1_hbm)


_sc_call = functools.partial(
    pl.kernel,
    mesh=plsc.VectorSubcoreMesh(core_axis_name="c", subcore_axis_name="s"),
    compiler_params=pltpu.CompilerParams(needs_layout_passes=False),
    out_type=[
        jax.ShapeDtypeStruct((N, HH), _f32),
        jax.ShapeDtypeStruct((N, HH), _f32),
    ],
    scratch_types=[
        pltpu.VMEM((N,), _f32),
        pltpu.VMEM((N,), _f32),
        pltpu.VMEM((B64, HH), _f32),
        pltpu.VMEM((B64, HH), _f32),
        pltpu.VMEM((GRP, BLK), _i32),
        pltpu.VMEM((GRP, BLK), _i32),
        pltpu.VMEM((GRP, BLK), _f32),
        pltpu.VMEM((GRP, B64), _i32),
        pltpu.VMEM((GRP, B64), _i32),
        pltpu.VMEM((GRP, B64), _f32),
        pltpu.VMEM((GRP, B64), _f32),
        pltpu.VMEM((640,), _f32),
        pltpu.VMEM((16,), _f32),
        pltpu.VMEM((16, 16), _f32),
        pltpu.SemaphoreType.DMA,
        pltpu.SemaphoreType.DMA,
        pltpu.SemaphoreType.DMA,
        pltpu.SemaphoreType.DMA,
        pltpu.SemaphoreType.DMA,
        pltpu.SemaphoreType.DMA,
        pltpu.SemaphoreType.DMA,
        pltpu.VMEM_SHARED((N, HH), _f32),
        pltpu.VMEM_SHARED((DEN_PAD,), _f32),
        pltpu.VMEM_SHARED((16, 16), _f32),
    ],
)(_sc_body)


def kernel(x, edge_index, batch, W1, a_s1, a_d1, b1, W2, a_s2, a_d2, b2,
           W3, a_s3, a_d3, b3, fcW1, fcb1, fcW2, fcb2):
  loop = jnp.arange(N, dtype=_i32)
  pad = jnp.zeros((E_PAD - E_TOT,), _i32)
  srcf = jnp.concatenate([edge_index[0].astype(_i32), loop, pad])
  dstf = jnp.concatenate([edge_index[1].astype(_i32), loop, pad])
  src = srcf.reshape(TILES, NBLK, BLK)
  dst = dstf.reshape(TILES, NBLK, BLK)
  src64 = srcf.reshape(TILES, NB64, B64)
  dst64 = dstf.reshape(TILES, NB64, B64)

  h0, h1, asv, adv = _layer1_call(x, W1, a_s1.reshape(1, H),
                                  a_d1.reshape(1, H))
  o0, o1 = _sc_call(asv.reshape(N), adv.reshape(N), src, dst,
                    src64, dst64, h0, h1)

  h0, h1, asv, adv = _layer_call(o0, o1, b1.reshape(1, H), W2,
                                 a_s2.reshape(1, H), a_d2.reshape(1, H))
  o0, o1 = _sc_call(asv.reshape(N), adv.reshape(N), src, dst,
                    src64, dst64, h0, h1)

  h0, h1, asv, adv = _layer_call(o0, o1, b2.reshape(1, H), W3,
                                 a_s3.reshape(1, H), a_d3.reshape(1, H))
  o0, o1 = _sc_call(asv.reshape(N), adv.reshape(N), src, dst,
                    src64, dst64, h0, h1)

  out = _readout_call(o0, o1, b3.reshape(1, H), batch.reshape(1, N),
                      fcW1, fcb1.reshape(1, 196), fcW2, fcb2.reshape(1, 10))
  return out
